# baseline (speedup 1.0000x reference)
import jax
import jax.numpy as jnp
from jax import lax
from jax.experimental import pallas as pl
from jax.experimental.pallas import tpu as pltpu
from jax.experimental.pallas import tpu_sc as plsc

_NF = 26
_NFP = 27
_NB = _NFP // 3
_D = 16
_W = _D // 2
_NP = (_NF * (_NF - 1)) // 2
_NPP = 328
_NC = 2
_NS = 16
_NW = _NC * _NS
_L = 16
_BS = 3
_TC = 128
_SEG = 8 * _TC

_THR = []
_acc = 0
for _i in range(_NB - 2):
    _acc += _NB - 1 - _i
    _THR.append(_acc)


def _pair_k(i, j):
    return 25 * i - (i * (i - 1)) // 2 + (j - i - 1)


def _st_k(i, j):
    return jnp.where(j >= _NF, _NP, _pair_k(i, jnp.minimum(j, _NF - 1)))


def _make_body(n):
    n_cols = n // _TC
    cols_per = n_cols // _NW
    n_groups = _TC // _L
    n_offdiag = (_NB * (_NB - 1)) // 2

    def body(x_hbm, o_hbm, x_v, o_v):
        wid = lax.axis_index("s") * _NC + lax.axis_index("c")
        col0 = wid * cols_per

        def zero_pad(i, carry):
            x_v[_NF, pl.ds(i * _L, _L)] = jnp.zeros((_L,), jnp.uint32)
            return carry

        lax.fori_loop(0, _SEG // _L, zero_pad, 0)

        def col_body(ci, carry):
            tc = col0 + ci
            pltpu.sync_copy(x_hbm.at[:, tc], x_v.at[pl.ds(0, _NF)])

            def group_body(g, carry2):
                b0 = g * _L

                def ld(f, w):
                    word = x_v[f, pl.ds(w * _TC + b0, _L)]
                    return plsc.bitcast(word, jnp.bfloat16)

                def st(k, acc):
                    lo, hi = plsc.unpack(
                        acc, format=plsc.PackFormat.INTERLEAVED)
                    o_v[k // 8, pl.ds((k % 8) * _TC + b0, _L)] = lo + hi

                def offdiag(t, c3):
                    bi = sum(((t >= thr).astype(jnp.int32) for thr in _THR),
                             jnp.int32(0))
                    bj = t - ((_NB - 1) * bi - (bi * (bi - 1)) // 2) + bi + 1
                    ib = _BS * bi
                    jb = _BS * bj
                    pa = [ld(ib + a, 0) for a in range(_BS)]
                    qa = [ld(jb + b, 0) for b in range(_BS)]
                    acc = [[pa[a] * qa[b] for b in range(_BS)]
                           for a in range(_BS)]
                    for w in range(1, _W):
                        pa = [ld(ib + a, w) for a in range(_BS)]
                        qa = [ld(jb + b, w) for b in range(_BS)]
                        for a in range(_BS):
                            for b in range(_BS):
                                acc[a][b] = acc[a][b] + pa[a] * qa[b]
                    for a in range(_BS):
                        for b in range(_BS):
                            st(_st_k(ib + a, jb + b), acc[a][b])
                    return c3

                lax.fori_loop(0, n_offdiag, offdiag, 0)

                def diag(t, c3):
                    ib = _BS * t
                    pa = [ld(ib + a, 0) for a in range(_BS)]
                    acc = {(a, b): pa[a] * pa[b]
                           for a in range(_BS) for b in range(a + 1, _BS)}
                    for w in range(1, _W):
                        pa = [ld(ib + a, w) for a in range(_BS)]
                        for a in range(_BS):
                            for b in range(a + 1, _BS):
                                acc[(a, b)] = acc[(a, b)] + pa[a] * pa[b]
                    for a in range(_BS):
                        for b in range(a + 1, _BS):
                            st(_st_k(ib + a, ib + b), acc[(a, b)])
                    return c3

                lax.fori_loop(0, _NB, diag, 0)
                return carry2

            lax.fori_loop(0, n_groups, group_body, 0)
            pltpu.sync_copy(o_v, o_hbm.at[:, tc])
            return carry

        lax.fori_loop(0, cols_per, col_body, 0)

    return body


def kernel(inputs):
    n = inputs.shape[0]
    n_cols = n // _TC
    xi = lax.bitcast_convert_type(inputs, jnp.uint32)

    def _rne(u):
        return (u + jnp.uint32(0x7FFF) + ((u >> 16) & jnp.uint32(1))) >> 16

    w = _rne(xi[:, :, 0::2]) | (_rne(xi[:, :, 1::2]) << 16)
    ut = w.transpose(1, 2, 0)
    xw = ut.reshape(_NF, _W, n_cols, _TC).transpose(0, 2, 1, 3)
    xw = xw.reshape(_NF, n_cols, _SEG)

    mesh = plsc.VectorSubcoreMesh(core_axis_name="c", subcore_axis_name="s",
                                  num_cores=_NC, num_subcores=_NS)
    f = pl.kernel(
        _make_body(n),
        out_type=jax.ShapeDtypeStruct((_NPP // 8, n_cols, _SEG), jnp.float32),
        mesh=mesh,
        scratch_types=[pltpu.VMEM((_NFP, _SEG), jnp.uint32),
                       pltpu.VMEM((_NPP // 8, _SEG), jnp.float32)],
        compiler_params=pltpu.CompilerParams(use_tc_tiling_on_sc=False,
                                             needs_layout_passes=False),
    )
    out = f(xw)
    y = out.reshape(_NPP // 8, n_cols, 8, _TC)
    y = y.transpose(0, 2, 1, 3).reshape(_NPP, n)
    return y[:_NP].T


# speedup vs baseline: 1.8640x; 1.8640x over previous
import jax
import jax.numpy as jnp
from jax import lax
from jax.experimental import pallas as pl
from jax.experimental.pallas import tpu as pltpu
from jax.experimental.pallas import tpu_sc as plsc

_NF = 26
_NFP = 27
_NB = _NFP // 3
_D = 16
_W = _D // 2
_NP = (_NF * (_NF - 1)) // 2
_NPP = 328
_NC = 2
_NS = 16
_NW = _NC * _NS
_L = 16
_BS = 3
_TC = 128
_SEG = 8 * _TC

_THR = []
_acc = 0
for _i in range(_NB - 2):
    _acc += _NB - 1 - _i
    _THR.append(_acc)


def _pair_k(i, j):
    return 25 * i - (i * (i - 1)) // 2 + (j - i - 1)


def _st_k(i, j):
    return jnp.where(j >= _NF, _NP, _pair_k(i, jnp.minimum(j, _NF - 1)))


def _make_body(n):
    n_cols = n // _TC
    cols_per = n_cols // _NW
    n_groups = _TC // _L
    n_offdiag = (_NB * (_NB - 1)) // 2

    def body(x_hbm, o_hbm, x_v, xp_v, o_v):
        wid = lax.axis_index("s") * _NC + lax.axis_index("c")
        col0 = wid * cols_per

        def zero_pad(i, carry):
            xp_v[_NF, pl.ds(i * _L, _L)] = jnp.zeros((_L,), jnp.uint32)
            return carry

        lax.fori_loop(0, _SEG // _L, zero_pad, 0)

        def col_body(ci, carry):
            tc = col0 + ci
            pltpu.sync_copy(x_hbm.at[:, tc], x_v)

            def pack_field(f, carry2):
                for w in range(_W):
                    row = f * 2 + w // 4
                    clo = ((2 * w) % 8) * _TC
                    chi = ((2 * w + 1) % 8) * _TC
                    for g in range(n_groups):
                        lo = x_v[row, pl.ds(clo + g * _L, _L)]
                        hi = x_v[row, pl.ds(chi + g * _L, _L)]
                        pk = plsc.pack(lo, hi,
                                       format=plsc.PackFormat.INTERLEAVED)
                        xp_v[f, pl.ds(w * _TC + g * _L, _L)] = \
                            plsc.bitcast(pk, jnp.uint32)
                return carry2

            lax.fori_loop(0, _NF, pack_field, 0)

            def group_body(g, carry2):
                b0 = g * _L

                def ld(f, w):
                    word = xp_v[f, pl.ds(w * _TC + b0, _L)]
                    return plsc.bitcast(word, jnp.bfloat16)

                def st(k, acc):
                    lo, hi = plsc.unpack(
                        acc, format=plsc.PackFormat.INTERLEAVED)
                    o_v[k // 8, pl.ds((k % 8) * _TC + b0, _L)] = lo + hi

                def offdiag(t, c3):
                    bi = sum(((t >= thr).astype(jnp.int32) for thr in _THR),
                             jnp.int32(0))
                    bj = t - ((_NB - 1) * bi - (bi * (bi - 1)) // 2) + bi + 1
                    ib = _BS * bi
                    jb = _BS * bj
                    pa = [ld(ib + a, 0) for a in range(_BS)]
                    qa = [ld(jb + b, 0) for b in range(_BS)]
                    acc = [[pa[a] * qa[b] for b in range(_BS)]
                           for a in range(_BS)]
                    for w in range(1, _W):
                        pa = [ld(ib + a, w) for a in range(_BS)]
                        qa = [ld(jb + b, w) for b in range(_BS)]
                        for a in range(_BS):
                            for b in range(_BS):
                                acc[a][b] = acc[a][b] + pa[a] * qa[b]
                    for a in range(_BS):
                        for b in range(_BS):
                            st(_st_k(ib + a, jb + b), acc[a][b])
                    return c3

                lax.fori_loop(0, n_offdiag, offdiag, 0)

                def diag(t, c3):
                    ib = _BS * t
                    pa = [ld(ib + a, 0) for a in range(_BS)]
                    acc = {(a, b): pa[a] * pa[b]
                           for a in range(_BS) for b in range(a + 1, _BS)}
                    for w in range(1, _W):
                        pa = [ld(ib + a, w) for a in range(_BS)]
                        for a in range(_BS):
                            for b in range(a + 1, _BS):
                                acc[(a, b)] = acc[(a, b)] + pa[a] * pa[b]
                    for a in range(_BS):
                        for b in range(a + 1, _BS):
                            st(_st_k(ib + a, ib + b), acc[(a, b)])
                    return c3

                lax.fori_loop(0, _NB, diag, 0)
                return carry2

            lax.fori_loop(0, n_groups, group_body, 0)
            pltpu.sync_copy(o_v, o_hbm.at[:, tc])
            return carry

        lax.fori_loop(0, cols_per, col_body, 0)

    return body


def kernel(inputs):
    n = inputs.shape[0]
    n_cols = n // _TC
    v = inputs.transpose(1, 2, 0)
    v = v.reshape(_NF, 2, 8, n_cols, _TC)
    v = v.transpose(0, 1, 3, 2, 4)
    xv = v.reshape(_NF * 2, n_cols, _SEG)

    mesh = plsc.VectorSubcoreMesh(core_axis_name="c", subcore_axis_name="s",
                                  num_cores=_NC, num_subcores=_NS)
    f = pl.kernel(
        _make_body(n),
        out_type=jax.ShapeDtypeStruct((_NPP // 8, n_cols, _SEG), jnp.float32),
        mesh=mesh,
        scratch_types=[pltpu.VMEM((_NF * 2, _SEG), jnp.float32),
                       pltpu.VMEM((_NFP, _SEG), jnp.uint32),
                       pltpu.VMEM((_NPP // 8, _SEG), jnp.float32)],
        compiler_params=pltpu.CompilerParams(use_tc_tiling_on_sc=False,
                                             needs_layout_passes=False),
    )
    out = f(xv)
    y = out.reshape(_NPP // 8, n_cols, 8, _TC)
    y = y.transpose(0, 2, 1, 3).reshape(_NPP, n)
    return y[:_NP].T


# speedup vs baseline: 2.0353x; 1.0919x over previous
import jax
import jax.numpy as jnp
from jax import lax
from jax.experimental import pallas as pl
from jax.experimental.pallas import tpu as pltpu
from jax.experimental.pallas import tpu_sc as plsc

_NF = 26
_NFP = 27
_NB = _NFP // 3
_D = 16
_W = _D // 2
_NP = (_NF * (_NF - 1)) // 2
_NPP = 328
_NC = 2
_NS = 16
_NW = _NC * _NS
_L = 16
_BS = 3
_TC = 128
_SEG = 8 * _TC

_THR = []
_acc = 0
for _i in range(_NB - 2):
    _acc += _NB - 1 - _i
    _THR.append(_acc)


def _pair_k(i, j):
    return 25 * i - (i * (i - 1)) // 2 + (j - i - 1)


def _st_k(i, j):
    return jnp.where(j >= _NF, _NP, _pair_k(i, jnp.minimum(j, _NF - 1)))


def _make_body(n):
    n_cols = n // _TC
    cols_per = n_cols // _NW
    n_groups = _TC // _L
    n_offdiag = (_NB * (_NB - 1)) // 2

    def body(x_hbm, o_hbm, x_v, xp_v, o_v, sem_in, sem_out):
        wid = lax.axis_index("s") * _NC + lax.axis_index("c")
        col0 = wid * cols_per

        def zero_pad(i, carry):
            xp_v[_NF, pl.ds(i * _L, _L)] = jnp.zeros((_L,), jnp.uint32)
            return carry

        lax.fori_loop(0, _SEG // _L, zero_pad, 0)

        pltpu.async_copy(x_hbm.at[:, col0], x_v, sem_in)

        def col_body(ci, carry):
            tc = col0 + ci
            pltpu.make_async_copy(x_hbm.at[:, tc], x_v, sem_in).wait()

            def pack_field(f, carry2):
                for w in range(_W):
                    row = f * 2 + w // 4
                    clo = ((2 * w) % 8) * _TC
                    chi = ((2 * w + 1) % 8) * _TC
                    for g in range(n_groups):
                        lo = x_v[row, pl.ds(clo + g * _L, _L)]
                        hi = x_v[row, pl.ds(chi + g * _L, _L)]
                        pk = plsc.pack(lo, hi,
                                       format=plsc.PackFormat.INTERLEAVED)
                        xp_v[f, pl.ds(w * _TC + g * _L, _L)] = \
                            plsc.bitcast(pk, jnp.uint32)
                return carry2

            lax.fori_loop(0, _NF, pack_field, 0)

            tc_next = col0 + jnp.minimum(ci + 1, cols_per - 1)
            pltpu.async_copy(x_hbm.at[:, tc_next], x_v, sem_in)

            @pl.when(ci > 0)
            def _wait_prev_out():
                pltpu.make_async_copy(
                    o_v, o_hbm.at[:, tc - 1], sem_out).wait()

            def group_body(g, carry2):
                b0 = g * _L

                def ld(f, w):
                    word = xp_v[f, pl.ds(w * _TC + b0, _L)]
                    return plsc.bitcast(word, jnp.bfloat16)

                def st(k, acc):
                    lo, hi = plsc.unpack(
                        acc, format=plsc.PackFormat.INTERLEAVED)
                    o_v[k // 8, pl.ds((k % 8) * _TC + b0, _L)] = lo + hi

                def offdiag(t, c3):
                    bi = sum(((t >= thr).astype(jnp.int32) for thr in _THR),
                             jnp.int32(0))
                    bj = t - ((_NB - 1) * bi - (bi * (bi - 1)) // 2) + bi + 1
                    ib = _BS * bi
                    jb = _BS * bj
                    pa = [ld(ib + a, 0) for a in range(_BS)]
                    qa = [ld(jb + b, 0) for b in range(_BS)]
                    acc = [[pa[a] * qa[b] for b in range(_BS)]
                           for a in range(_BS)]
                    for w in range(1, _W):
                        pa = [ld(ib + a, w) for a in range(_BS)]
                        qa = [ld(jb + b, w) for b in range(_BS)]
                        for a in range(_BS):
                            for b in range(_BS):
                                acc[a][b] = acc[a][b] + pa[a] * qa[b]
                    for a in range(_BS):
                        for b in range(_BS):
                            st(_st_k(ib + a, jb + b), acc[a][b])
                    return c3

                lax.fori_loop(0, n_offdiag, offdiag, 0)

                def diag(t, c3):
                    ib = _BS * t
                    pa = [ld(ib + a, 0) for a in range(_BS)]
                    acc = {(a, b): pa[a] * pa[b]
                           for a in range(_BS) for b in range(a + 1, _BS)}
                    for w in range(1, _W):
                        pa = [ld(ib + a, w) for a in range(_BS)]
                        for a in range(_BS):
                            for b in range(a + 1, _BS):
                                acc[(a, b)] = acc[(a, b)] + pa[a] * pa[b]
                    for a in range(_BS):
                        for b in range(a + 1, _BS):
                            st(_st_k(ib + a, ib + b), acc[(a, b)])
                    return c3

                lax.fori_loop(0, _NB, diag, 0)
                return carry2

            lax.fori_loop(0, n_groups, group_body, 0)
            pltpu.async_copy(o_v, o_hbm.at[:, tc], sem_out)
            return carry

        lax.fori_loop(0, cols_per, col_body, 0)
        pltpu.make_async_copy(
            o_v, o_hbm.at[:, col0 + cols_per - 1], sem_out).wait()
        pltpu.make_async_copy(
            x_hbm.at[:, col0 + cols_per - 1], x_v, sem_in).wait()

    return body


def kernel(inputs):
    n = inputs.shape[0]
    n_cols = n // _TC
    v = inputs.transpose(1, 2, 0)
    v = v.reshape(_NF, 2, 8, n_cols, _TC)
    v = v.transpose(0, 1, 3, 2, 4)
    xv = v.reshape(_NF * 2, n_cols, _SEG)

    mesh = plsc.VectorSubcoreMesh(core_axis_name="c", subcore_axis_name="s",
                                  num_cores=_NC, num_subcores=_NS)
    f = pl.kernel(
        _make_body(n),
        out_type=jax.ShapeDtypeStruct((_NPP // 8, n_cols, _SEG), jnp.float32),
        mesh=mesh,
        scratch_types=[pltpu.VMEM((_NF * 2, _SEG), jnp.float32),
                       pltpu.VMEM((_NFP, _SEG), jnp.uint32),
                       pltpu.VMEM((_NPP // 8, _SEG), jnp.float32),
                       pltpu.SemaphoreType.DMA,
                       pltpu.SemaphoreType.DMA],
        compiler_params=pltpu.CompilerParams(use_tc_tiling_on_sc=False,
                                             needs_layout_passes=False),
    )
    out = f(xv)
    y = out.reshape(_NPP // 8, n_cols, 8, _TC)
    y = y.transpose(0, 2, 1, 3).reshape(_NPP, n)
    return y[:_NP].T
